# baseline (speedup 1.0000x reference)
import functools

import jax
import jax.numpy as jnp
from jax import lax
from jax.experimental import pallas as pl
from jax.experimental.pallas import tpu as pltpu
from jax.experimental.pallas import tpu_sc as plsc

_fn_cache = {}


def _build_gather(b, f, dim):
  info = plsc.get_sparse_core_info()
  nw = info.num_cores * info.num_subcores
  rpw = b // nw
  dt_n = dim // 8
  cb = 64
  n_chunk = rpw // cb
  cf = cb * f

  mesh = plsc.VectorSubcoreMesh(core_axis_name="c", subcore_axis_name="s")

  @functools.partial(
      pl.kernel,
      mesh=mesh,
      out_type=jax.ShapeDtypeStruct((f, dt_n, b // 128, 8, 128), jnp.float32),
      scratch_types=[
          pltpu.VMEM((rpw, f), jnp.int32),
          [pltpu.VMEM((cf, dim // 2), jnp.float32) for _ in range(2)],
          [pltpu.VMEM((cf, dim // 2), jnp.float32) for _ in range(2)],
          pltpu.VMEM((f, dt_n, 8, 128), jnp.float32),
          pltpu.SemaphoreType.DMA,
          pltpu.SemaphoreType.DMA,
      ],
      compiler_params=pltpu.CompilerParams(
          use_tc_tiling_on_sc=False, needs_layout_passes=False
      ),
  )
  def gather(idx_hbm, ta_hbm, tb_hbm, out_hbm, idx_v, gabufs, gbbufs, tbuf, gsem, ssem):
    wid = lax.axis_index("s") * info.num_cores + lax.axis_index("c")
    base = pl.multiple_of(wid * rpw, 8)
    bt0 = wid * (rpw // 128)
    pltpu.sync_copy(idx_hbm.at[pl.ds(base, rpw)], idx_v)

    i26 = lax.iota(jnp.int32, 16) * f

    def start_gathers(i):
      def row(r, _):
        pltpu.async_copy(
            ta_hbm.at[idx_v.at[i * cb + r]],
            gabufs[i % 2].at[pl.ds(r * f, f)],
            gsem,
        )
        pltpu.async_copy(
            tb_hbm.at[idx_v.at[i * cb + r]],
            gbbufs[i % 2].at[pl.ds(r * f, f)],
            gsem,
        )
        return 0

      lax.fori_loop(0, cb, row, 0)

    def gathers_done(i):
      pltpu.make_async_copy(
          out_hbm.at[:, :, 0, :, pl.ds(0, cb)],
          tbuf.at[:, :, :, pl.ds(0, cb)],
          gsem,
      ).wait()

    dvs = [jnp.full((16,), dd, jnp.int32) for dd in range(dim // 2)]

    def transpose_chunk(i):
      ga, gb = gabufs[i % 2], gbbufs[i % 2]
      half = (i % 2) * cb

      def f_iter(ff, _):
        tview = tbuf.at[ff]
        for blg in range(cb // 16):
          jv = i26 + (blg * 16 * f + ff)
          vals = [plsc.load_gather(ga, [jv, dvs[dd]]) for dd in range(dim // 2)]
          vals += [plsc.load_gather(gb, [jv, dvs[dd]]) for dd in range(dim // 2)]
          for dd in range(dim):
            tview[dd // 8, dd % 8, pl.ds(half + blg * 16, 16)] = vals[dd]
        return 0

      lax.fori_loop(0, f, f_iter, 0)

    def store_copies(bt):
      return [
          pltpu.make_async_copy(
              tbuf.at[ff, dt], out_hbm.at[ff, dt, bt0 + bt], ssem
          )
          for ff in range(f)
          for dt in range(dt_n)
      ]

    start_gathers(0)
    for i in range(n_chunk):
      gathers_done(i)
      if i + 1 < n_chunk:
        start_gathers(i + 1)
      if i % 2 == 0 and i > 0:
        pltpu.make_async_copy(out_hbm.at[:, :, 0], tbuf, ssem).wait()
      transpose_chunk(i)
      if i % 2 == 1:
        for c in store_copies(i // 2):
          c.start()
    pltpu.make_async_copy(out_hbm.at[:, :, 0], tbuf, ssem).wait()

  return gather


def kernel(indices, weight):
  b, f = indices.shape
  v, dim = weight.shape
  key = (b, f, v, dim)
  if key not in _fn_cache:
    _fn_cache[key] = _build_gather(b, f, dim)
  h = dim // 2
  out5 = _fn_cache[key](indices.astype(jnp.int32), weight[:, :h], weight[:, h:])
  return jnp.transpose(out5, (2, 4, 0, 1, 3)).reshape(b, f, dim)


# speedup vs baseline: 2.2357x; 2.2357x over previous
import functools

import jax
import jax.numpy as jnp
from jax import lax
from jax.experimental import pallas as pl
from jax.experimental.pallas import tpu as pltpu
from jax.experimental.pallas import tpu_sc as plsc

_fn_cache = {}


def _build_gather(b, f, dim):
  info = plsc.get_sparse_core_info()
  nw = info.num_cores * info.num_subcores
  rpw = b // nw
  dt_n = dim // 8
  cb = 64
  n_chunk = rpw // cb
  cf = cb * f

  mesh = plsc.VectorSubcoreMesh(core_axis_name="c", subcore_axis_name="s")

  @functools.partial(
      pl.kernel,
      mesh=mesh,
      out_type=jax.ShapeDtypeStruct((f, dt_n, b // 128, 8, 128), jnp.float32),
      scratch_types=[
          pltpu.VMEM((rpw, f), jnp.int32),
          [pltpu.VMEM((cf, dim), jnp.float32) for _ in range(2)],
          pltpu.VMEM((f, dt_n, 8, 128), jnp.float32),
          pltpu.SemaphoreType.DMA,
          pltpu.SemaphoreType.DMA,
      ],
      compiler_params=pltpu.CompilerParams(
          use_tc_tiling_on_sc=False, needs_layout_passes=False
      ),
  )
  def gather(idx_hbm, table_hbm, out_hbm, idx_v, gbufs, tbuf, gsem, ssem):
    wid = lax.axis_index("s") * info.num_cores + lax.axis_index("c")
    base = pl.multiple_of(wid * rpw, 8)
    bt0 = wid * (rpw // 128)
    pltpu.sync_copy(idx_hbm.at[pl.ds(base, rpw)], idx_v)

    i26 = lax.iota(jnp.int32, 16) * f

    def row_copy(i, r):
      return pltpu.make_async_copy(
          table_hbm.at[idx_v.at[i * cb + r]],
          gbufs[i % 2].at[pl.ds(r * f, f)],
          gsem,
      )

    def start_gathers(i):
      lax.fori_loop(0, cb, lambda r, _: (row_copy(i, r).start(), 0)[1], 0)

    def gathers_done(i):
      pltpu.make_async_copy(
          out_hbm.at[:, :, 0, :, pl.ds(0, cb)],
          tbuf.at[:, :, :, pl.ds(0, cb)],
          gsem,
      ).wait()

    dvs = [jnp.full((16,), dd, jnp.int32) for dd in range(dim)]

    def transpose_chunk(i):
      gbuf = gbufs[i % 2]
      half = (i % 2) * cb

      def f_iter(ff, _):
        tview = tbuf.at[ff]
        for blg in range(cb // 16):
          jv = i26 + (blg * 16 * f + ff)
          vals = [plsc.load_gather(gbuf, [jv, dvs[dd]]) for dd in range(dim)]
          for dd in range(dim):
            tview[dd // 8, dd % 8, pl.ds(half + blg * 16, 16)] = vals[dd]
        return 0

      lax.fori_loop(0, f, f_iter, 0)

    def store_copies(bt):
      return [
          pltpu.make_async_copy(
              tbuf.at[ff, dt], out_hbm.at[ff, dt, bt0 + bt], ssem
          )
          for ff in range(f)
          for dt in range(dt_n)
      ]

    start_gathers(0)
    for i in range(n_chunk):
      gathers_done(i)
      if i + 1 < n_chunk:
        start_gathers(i + 1)
      if i % 2 == 0 and i > 0:
        pltpu.make_async_copy(out_hbm.at[:, :, 0], tbuf, ssem).wait()
      transpose_chunk(i)
      if i % 2 == 1:
        for c in store_copies(i // 2):
          c.start()
    pltpu.make_async_copy(out_hbm.at[:, :, 0], tbuf, ssem).wait()

  return gather


def kernel(indices, weight):
  b, f = indices.shape
  v, dim = weight.shape
  key = (b, f, v, dim)
  if key not in _fn_cache:
    _fn_cache[key] = _build_gather(b, f, dim)
  out5 = _fn_cache[key](indices.astype(jnp.int32), weight)
  return jnp.transpose(out5, (2, 4, 0, 1, 3)).reshape(b, f, dim)
